# baseline (speedup 1.0000x reference)
import jax
import jax.numpy as jnp
from jax import lax
from jax.experimental import pallas as pl
from jax.experimental.pallas import tpu as pltpu
from jax.experimental.pallas import tpu_sc as plsc

_N_NODES = 100000
_N_EDGES = 6400000
_STEPS = 3

_NC = 2
_NS = 16
_NW = _NC * _NS

_NP = 102400
_NPW = _NP // _NW

_CH = 1024
_EW = 200704
_EP = _EW * _NW
_NCH = _EW // _CH
_NB = 4


def _scatter_body(p_hbm, src_hbm, dst_hbm, probs_hbm, out_hbm,
                  srcb0, srcb1, srcb2, srcb3, dstb0, dstb1, dstb2, dstb3,
                  pb0, pb1, pb2, pb3, gb0, gb1, gb2, gb3, acc, p_sh,
                  lsem0, lsem1, lsem2, lsem3, gsem0, gsem1, gsem2, gsem3,
                  stsem):
  c = lax.axis_index("c")
  s = lax.axis_index("s")
  wid = s * _NC + c
  srcb = (srcb0, srcb1, srcb2, srcb3)
  dstb = (dstb0, dstb1, dstb2, dstb3)
  pb = (pb0, pb1, pb2, pb3)
  gb = (gb0, gb1, gb2, gb3)
  lsems = (lsem0, lsem1, lsem2, lsem3)
  gsems = (gsem0, gsem1, gsem2, gsem3)

  zero16 = jnp.zeros((16,), jnp.float32)

  def zloop(i, carry):
    acc[pl.ds(i * 16, 16)] = zero16
    return carry

  nps = _NP // _NS
  pltpu.async_copy(p_hbm.at[pl.ds(s * nps, nps)],
                   p_sh.at[pl.ds(s * nps, nps)], stsem)
  lax.fori_loop(0, _NP // 16, zloop, 0, unroll=8)
  pltpu.make_async_copy(p_hbm.at[pl.ds(0, nps)],
                        p_sh.at[pl.ds(0, nps)], stsem).wait()
  plsc.subcore_barrier()

  def issue_linear(ci, b):
    base = wid * _EW + ci * _CH
    pltpu.async_copy(src_hbm.at[pl.ds(base, _CH)], srcb[b], lsems[b])
    pltpu.async_copy(probs_hbm.at[pl.ds(base, _CH)], pb[b], lsems[b])
    pltpu.async_copy(dst_hbm.at[pl.ds(base, _CH)], dstb[b], lsems[b])

  def wait_linear(b):
    pltpu.make_async_copy(src_hbm.at[pl.ds(0, _CH)], srcb[b],
                          lsems[b]).wait()
    pltpu.make_async_copy(probs_hbm.at[pl.ds(0, _CH)], pb[b],
                          lsems[b]).wait()
    pltpu.make_async_copy(dst_hbm.at[pl.ds(0, _CH)], dstb[b],
                          lsems[b]).wait()

  def issue_gather(b):
    tab = p_sh if b else p_hbm
    pltpu.async_copy(tab.at[srcb[b]], gb[b], gsems[b])

  def wait_gather(b):
    tab = p_sh if b else p_hbm
    pltpu.make_async_copy(tab.at[srcb[b]], gb[b], gsems[b]).wait()

  def compute(b):
    def inner(j, icarry):
      sl = pl.ds(j * 16, 16)
      plsc.addupdate_scatter(acc, [dstb[b][sl]], gb[b][sl] * pb[b][sl])
      return icarry

    lax.fori_loop(0, _CH // 16, inner, 0, unroll=4)

  issue_linear(0, 0)
  issue_linear(1, 1)
  issue_linear(2, 2)
  wait_linear(0)
  issue_gather(0)

  def quad(k, carry):
    for b in range(_NB):
      ci = _NB * k + b

      @pl.when(ci + 3 < _NCH)
      def _():
        issue_linear(ci + 3, (b + 3) % _NB)

      @pl.when(ci + 1 < _NCH)
      def _():
        wait_linear((b + 1) % _NB)
        issue_gather((b + 1) % _NB)

      wait_gather(b)
      compute(b)
    return carry

  lax.fori_loop(0, _NCH // _NB, quad, 0)
  pltpu.sync_copy(acc, out_hbm.at[wid])


def _update_body(partials_hbm, cum_hbm, p_out, cum_out, fin_out,
                 rows, cumb, pbuf, finb):
  c = lax.axis_index("c")
  s = lax.axis_index("s")
  wid = s * _NC + c
  base = wid * _NPW

  pltpu.sync_copy(partials_hbm.at[:, pl.ds(base, _NPW)], rows)
  pltpu.sync_copy(cum_hbm.at[pl.ds(base, _NPW)], cumb)

  def red(j, carry):
    sl = pl.ds(j * 16, 16)
    d = rows[0, sl]
    for r in range(1, _NW):
      d = d + rows[r, sl]
    cm = cumb[sl]
    infl = jnp.exp(-d)
    pt = cm * (1.0 - infl)
    cn = cm * (1.0 - pt)
    pbuf[sl] = pt
    cumb[sl] = cn
    finb[sl] = 1.0 - cn
    return carry

  lax.fori_loop(0, _NPW // 16, red, 0)

  pltpu.sync_copy(pbuf, p_out.at[pl.ds(base, _NPW)])
  pltpu.sync_copy(cumb, cum_out.at[pl.ds(base, _NPW)])
  pltpu.sync_copy(finb, fin_out.at[pl.ds(base, _NPW)])


def _build_kernels():
  mesh = plsc.VectorSubcoreMesh(core_axis_name="c", subcore_axis_name="s")
  f32 = jnp.float32
  scatter = pl.kernel(
      _scatter_body,
      out_type=jax.ShapeDtypeStruct((_NW, _NP), f32),
      mesh=mesh,
      scratch_types=(
          [pltpu.VMEM((_CH,), jnp.int32)] * (2 * _NB)
          + [pltpu.VMEM((_CH,), f32)] * (2 * _NB)
          + [pltpu.VMEM((_NP,), f32)]
          + [pltpu.VMEM_SHARED((_NP,), f32)]
          + [pltpu.SemaphoreType.DMA] * 9
      ),
      compiler_params=pltpu.CompilerParams(needs_layout_passes=False),
      name="icapprox_scatter",
  )
  update = pl.kernel(
      _update_body,
      out_type=(
          jax.ShapeDtypeStruct((_NP,), f32),
          jax.ShapeDtypeStruct((_NP,), f32),
          jax.ShapeDtypeStruct((_NP,), f32),
      ),
      mesh=mesh,
      scratch_types=[
          pltpu.VMEM((_NW, _NPW), f32),
          pltpu.VMEM((_NPW,), f32),
          pltpu.VMEM((_NPW,), f32),
          pltpu.VMEM((_NPW,), f32),
      ],
      name="icapprox_update",
  )
  return scatter, update


def kernel(prior_probs, edge_index, edge_probs):
  pad_e = _EP - _N_EDGES
  src = jnp.pad(edge_index[0], (0, pad_e))
  dst = jnp.pad(edge_index[1], (0, pad_e))
  probs = jnp.pad(edge_probs, (0, pad_e))
  p = jnp.pad(prior_probs, (0, _NP - _N_NODES))
  cum = 1.0 - p
  scatter, update = _build_kernels()
  fin = None
  for _ in range(_STEPS):
    partials = scatter(p, src, dst, probs)
    p, cum, fin = update(partials, cum)
  return fin[:_N_NODES].reshape(-1, 1)


# speedup vs baseline: 1.4617x; 1.4617x over previous
import jax
import jax.numpy as jnp
from jax import lax
from jax.experimental import pallas as pl
from jax.experimental.pallas import tpu as pltpu
from jax.experimental.pallas import tpu_sc as plsc

_N_NODES = 100000
_N_EDGES = 6400000
_STEPS = 3

_NC = 2
_NS = 16
_NW = _NC * _NS

_NP = 102400
_NPW = _NP // _NW

_CH = 1024
_EW = 200704
_EP = _EW * _NW
_NCH = _EW // _CH
_NB = 4


def _scatter_body(p_hbm, src_hbm, dst_hbm, probs_hbm, out_hbm,
                  srcb0, srcb1, srcb2, srcb3, dstb0, dstb1, dstb2, dstb3,
                  pb0, pb1, pb2, pb3, gb0, gb1, gb2, gb3, acc, p_sh,
                  lsem0, lsem1, lsem2, lsem3, gsem0, gsem1, gsem2, gsem3,
                  stsem):
  c = lax.axis_index("c")
  s = lax.axis_index("s")
  wid = s * _NC + c
  srcb = (srcb0, srcb1, srcb2, srcb3)
  dstb = (dstb0, dstb1, dstb2, dstb3)
  pb = (pb0, pb1, pb2, pb3)
  gb = (gb0, gb1, gb2, gb3)
  lsems = (lsem0, lsem1, lsem2, lsem3)
  gsems = (gsem0, gsem1, gsem2, gsem3)

  zero16 = jnp.zeros((16,), jnp.float32)

  def zloop(i, carry):
    acc[pl.ds(i * 16, 16)] = zero16
    return carry

  nps = _NP // _NS
  pltpu.async_copy(p_hbm.at[pl.ds(s * nps, nps)],
                   p_sh.at[pl.ds(s * nps, nps)], stsem)
  lax.fori_loop(0, _NP // 16, zloop, 0, unroll=8)
  pltpu.make_async_copy(p_hbm.at[pl.ds(0, nps)],
                        p_sh.at[pl.ds(0, nps)], stsem).wait()
  plsc.subcore_barrier()

  def issue_linear(ci, b):
    base = wid * _EW + ci * _CH
    pltpu.async_copy(src_hbm.at[pl.ds(base, _CH)], srcb[b], lsems[b])
    pltpu.async_copy(probs_hbm.at[pl.ds(base, _CH)], pb[b], lsems[b])
    pltpu.async_copy(dst_hbm.at[pl.ds(base, _CH)], dstb[b], lsems[b])

  def wait_linear(b):
    pltpu.make_async_copy(src_hbm.at[pl.ds(0, _CH)], srcb[b],
                          lsems[b]).wait()
    pltpu.make_async_copy(probs_hbm.at[pl.ds(0, _CH)], pb[b],
                          lsems[b]).wait()
    pltpu.make_async_copy(dst_hbm.at[pl.ds(0, _CH)], dstb[b],
                          lsems[b]).wait()

  def issue_gather(b):
    tab = p_sh
    pltpu.async_copy(tab.at[srcb[b]], gb[b], gsems[b])

  def wait_gather(b):
    tab = p_sh
    pltpu.make_async_copy(tab.at[srcb[b]], gb[b], gsems[b]).wait()

  def compute(b):
    def inner(j, icarry):
      sl = pl.ds(j * 16, 16)
      plsc.addupdate_scatter(acc, [dstb[b][sl]], gb[b][sl] * pb[b][sl])
      return icarry

    lax.fori_loop(0, _CH // 16, inner, 0, unroll=4)

  issue_linear(0, 0)
  issue_linear(1, 1)
  issue_linear(2, 2)
  wait_linear(0)
  issue_gather(0)

  def quad(k, carry):
    for b in range(_NB):
      ci = _NB * k + b

      @pl.when(ci + 3 < _NCH)
      def _():
        issue_linear(ci + 3, (b + 3) % _NB)

      @pl.when(ci + 1 < _NCH)
      def _():
        wait_linear((b + 1) % _NB)
        issue_gather((b + 1) % _NB)

      wait_gather(b)
      compute(b)
    return carry

  lax.fori_loop(0, _NCH // _NB, quad, 0)
  pltpu.sync_copy(acc, out_hbm.at[wid])


def _update_body(partials_hbm, cum_hbm, p_out, cum_out, fin_out,
                 rows, cumb, pbuf, finb):
  c = lax.axis_index("c")
  s = lax.axis_index("s")
  wid = s * _NC + c
  base = wid * _NPW

  pltpu.sync_copy(partials_hbm.at[:, pl.ds(base, _NPW)], rows)
  pltpu.sync_copy(cum_hbm.at[pl.ds(base, _NPW)], cumb)

  def red(j, carry):
    sl = pl.ds(j * 16, 16)
    d = rows[0, sl]
    for r in range(1, _NW):
      d = d + rows[r, sl]
    cm = cumb[sl]
    infl = jnp.exp(-d)
    pt = cm * (1.0 - infl)
    cn = cm * (1.0 - pt)
    pbuf[sl] = pt
    cumb[sl] = cn
    finb[sl] = 1.0 - cn
    return carry

  lax.fori_loop(0, _NPW // 16, red, 0)

  pltpu.sync_copy(pbuf, p_out.at[pl.ds(base, _NPW)])
  pltpu.sync_copy(cumb, cum_out.at[pl.ds(base, _NPW)])
  pltpu.sync_copy(finb, fin_out.at[pl.ds(base, _NPW)])


def _build_kernels():
  mesh = plsc.VectorSubcoreMesh(core_axis_name="c", subcore_axis_name="s")
  f32 = jnp.float32
  scatter = pl.kernel(
      _scatter_body,
      out_type=jax.ShapeDtypeStruct((_NW, _NP), f32),
      mesh=mesh,
      scratch_types=(
          [pltpu.VMEM((_CH,), jnp.int32)] * (2 * _NB)
          + [pltpu.VMEM((_CH,), f32)] * (2 * _NB)
          + [pltpu.VMEM((_NP,), f32)]
          + [pltpu.VMEM_SHARED((_NP,), f32)]
          + [pltpu.SemaphoreType.DMA] * 9
      ),
      compiler_params=pltpu.CompilerParams(needs_layout_passes=False),
      name="icapprox_scatter",
  )
  update = pl.kernel(
      _update_body,
      out_type=(
          jax.ShapeDtypeStruct((_NP,), f32),
          jax.ShapeDtypeStruct((_NP,), f32),
          jax.ShapeDtypeStruct((_NP,), f32),
      ),
      mesh=mesh,
      scratch_types=[
          pltpu.VMEM((_NW, _NPW), f32),
          pltpu.VMEM((_NPW,), f32),
          pltpu.VMEM((_NPW,), f32),
          pltpu.VMEM((_NPW,), f32),
      ],
      name="icapprox_update",
  )
  return scatter, update


def kernel(prior_probs, edge_index, edge_probs):
  pad_e = _EP - _N_EDGES
  src = jnp.pad(edge_index[0], (0, pad_e))
  dst = jnp.pad(edge_index[1], (0, pad_e))
  probs = jnp.pad(edge_probs, (0, pad_e))
  p = jnp.pad(prior_probs, (0, _NP - _N_NODES))
  cum = 1.0 - p
  scatter, update = _build_kernels()
  fin = None
  for _ in range(_STEPS):
    partials = scatter(p, src, dst, probs)
    p, cum, fin = update(partials, cum)
  return fin[:_N_NODES].reshape(-1, 1)
